# Initial kernel scaffold; baseline (speedup 1.0000x reference)
import jax
import jax.numpy as jnp
from jax.experimental import pallas as pl


def kernel(seq1, seq2, adj, msk, samp_bias1, samp_bias2, W_fc, gcn_bias, prelu_a, bilin_W, bilin_b):
    raise NotImplementedError("write your pallas kernel here")


# speedup vs baseline: 1.5051x; 1.5051x over previous
import jax
import jax.numpy as jnp
from jax.experimental import pallas as pl
from jax.experimental.pallas import tpu as pltpu


def _fts_kernel(s1_ref, s2_ref, w_ref, out_ref):
    nh = w_ref.shape[1]
    w = w_ref[...].astype(jnp.bfloat16)
    f1 = jnp.dot(s1_ref[...].astype(jnp.bfloat16), w,
                 preferred_element_type=jnp.float32)
    f2 = jnp.dot(s2_ref[...].astype(jnp.bfloat16), w,
                 preferred_element_type=jnp.float32)
    out_ref[:, :nh] = f1.astype(jnp.bfloat16)
    out_ref[:, nh:] = f2.astype(jnp.bfloat16)


def _gcn_kernel(adj_ref, fts_ref, mskc_ref, bias2_ref, a_ref,
                h_ref, csum_ref):
    i = pl.program_id(0)
    nh = csum_ref.shape[1]
    ab = adj_ref[0].astype(jnp.bfloat16)
    acc = jnp.dot(ab, fts_ref[...],
                  preferred_element_type=jnp.float32)
    out = acc + bias2_ref[...]
    a = a_ref[0, 0]
    h = jnp.where(out >= 0, out, a * out)
    h_ref[...] = h

    @pl.when(i == 0)
    def _():
        csum_ref[...] = jnp.zeros_like(csum_ref)

    hm = h[:, :nh] * mskc_ref[...]
    csum_ref[...] += jnp.sum(hm, axis=0, keepdims=True)


def _disc_kernel(h_ref, csum_ref, msk_ref, wt_ref, b_ref,
                 sb1_ref, sb2_ref, o1_ref, o2_ref):
    nh = csum_ref.shape[1]
    smsk = jnp.sum(msk_ref[...])
    c = jax.nn.sigmoid(csum_ref[...] / smsk)
    v = jnp.dot(c, wt_ref[...],
                preferred_element_type=jnp.float32)
    b = b_ref[0, 0]
    h = h_ref[...]
    sc1 = jnp.sum(h[:, :nh] * v, axis=1, keepdims=True)
    sc2 = jnp.sum(h[:, nh:] * v, axis=1, keepdims=True)
    o1_ref[...] = sc1 + b + sb1_ref[...]
    o2_ref[...] = sc2 + b + sb2_ref[...]


def kernel(seq1, seq2, adj, msk, samp_bias1, samp_bias2,
           W_fc, gcn_bias, prelu_a, bilin_W, bilin_b):
    n = adj.shape[1]
    nh = W_fc.shape[1]
    nin = W_fc.shape[0]

    tb = 2000
    tm = 200
    td = 2000

    s1 = seq1[0]
    s2 = seq2[0]

    fts = pl.pallas_call(
        _fts_kernel,
        grid=(n // tb,),
        in_specs=[
            pl.BlockSpec((tb, nin), lambda i: (i, 0)),
            pl.BlockSpec((tb, nin), lambda i: (i, 0)),
            pl.BlockSpec((nin, nh), lambda i: (0, 0)),
        ],
        out_specs=pl.BlockSpec((tb, 2 * nh), lambda i: (i, 0)),
        out_shape=jax.ShapeDtypeStruct((n, 2 * nh), jnp.bfloat16),
    )(s1, s2, W_fc)

    bias2 = jnp.concatenate([gcn_bias, gcn_bias]).reshape(1, 2 * nh)
    a2 = prelu_a.reshape(1, 1)
    mskc = msk.reshape(n, 1)

    h, csum = pl.pallas_call(
        _gcn_kernel,
        grid=(n // tm,),
        in_specs=[
            pl.BlockSpec((1, tm, n), lambda i: (0, i, 0)),
            pl.BlockSpec((n, 2 * nh), lambda i: (0, 0)),
            pl.BlockSpec((tm, 1), lambda i: (i, 0)),
            pl.BlockSpec((1, 2 * nh), lambda i: (0, 0)),
            pl.BlockSpec((1, 1), lambda i: (0, 0)),
        ],
        out_specs=[
            pl.BlockSpec((tm, 2 * nh), lambda i: (i, 0)),
            pl.BlockSpec((1, nh), lambda i: (0, 0)),
        ],
        out_shape=[
            jax.ShapeDtypeStruct((n, 2 * nh), jnp.float32),
            jax.ShapeDtypeStruct((1, nh), jnp.float32),
        ],
        compiler_params=pltpu.CompilerParams(
            vmem_limit_bytes=60 * 1024 * 1024,
        ),
    )(adj, fts, mskc, bias2, a2)

    wt = bilin_W.T
    b2 = bilin_b.reshape(1, 1)
    sb1 = samp_bias1.reshape(n, 1)
    sb2 = samp_bias2.reshape(n, 1)

    o1, o2 = pl.pallas_call(
        _disc_kernel,
        grid=(n // td,),
        in_specs=[
            pl.BlockSpec((td, 2 * nh), lambda i: (i, 0)),
            pl.BlockSpec((1, nh), lambda i: (0, 0)),
            pl.BlockSpec((1, n), lambda i: (0, 0)),
            pl.BlockSpec((nh, nh), lambda i: (0, 0)),
            pl.BlockSpec((1, 1), lambda i: (0, 0)),
            pl.BlockSpec((td, 1), lambda i: (i, 0)),
            pl.BlockSpec((td, 1), lambda i: (i, 0)),
        ],
        out_specs=[
            pl.BlockSpec((td, 1), lambda i: (i, 0)),
            pl.BlockSpec((td, 1), lambda i: (i, 0)),
        ],
        out_shape=[
            jax.ShapeDtypeStruct((n, 1), jnp.float32),
            jax.ShapeDtypeStruct((n, 1), jnp.float32),
        ],
    )(h, csum, msk, wt, b2, sb1, sb2)

    logits = jnp.concatenate([o1[:, 0], o2[:, 0]])[None, :]
    return logits


# speedup vs baseline: 1.5364x; 1.0208x over previous
import jax
import jax.numpy as jnp
from jax.experimental import pallas as pl
from jax.experimental.pallas import tpu as pltpu


def _fts_kernel(s1_ref, s2_ref, w_ref, out_ref):
    nh = w_ref.shape[1]
    w = w_ref[...].astype(jnp.bfloat16)
    f1 = jnp.dot(s1_ref[...].astype(jnp.bfloat16), w,
                 preferred_element_type=jnp.float32)
    f2 = jnp.dot(s2_ref[...].astype(jnp.bfloat16), w,
                 preferred_element_type=jnp.float32)
    out_ref[:, :nh] = f1.astype(jnp.bfloat16)
    out_ref[:, nh:] = f2.astype(jnp.bfloat16)


def _gcn_kernel(adj_ref, fts_ref, mskc_ref, bias2_ref, a_ref,
                h_ref, csum_ref):
    i = pl.program_id(0)
    nh = csum_ref.shape[1]
    ab = adj_ref[0].astype(jnp.bfloat16)
    acc = jnp.dot(ab, fts_ref[...],
                  preferred_element_type=jnp.float32)
    out = acc + bias2_ref[...]
    a = a_ref[0, 0]
    h = jnp.where(out >= 0, out, a * out)
    h_ref[...] = h.astype(jnp.bfloat16)

    @pl.when(i == 0)
    def _():
        csum_ref[...] = jnp.zeros_like(csum_ref)

    hm = h[:, :nh] * mskc_ref[...]
    csum_ref[...] += jnp.sum(hm, axis=0, keepdims=True)


def _disc_kernel(h_ref, csum_ref, msk_ref, wt_ref, b_ref,
                 sb1_ref, sb2_ref, o1_ref, o2_ref):
    nh = csum_ref.shape[1]
    smsk = jnp.sum(msk_ref[...])
    c = jax.nn.sigmoid(csum_ref[...] / smsk)
    v = jnp.dot(c, wt_ref[...],
                preferred_element_type=jnp.float32)
    b = b_ref[0, 0]
    h = h_ref[...].astype(jnp.float32)
    sc1 = jnp.sum(h[:, :nh] * v, axis=1, keepdims=True)
    sc2 = jnp.sum(h[:, nh:] * v, axis=1, keepdims=True)
    o1_ref[...] = sc1 + b + sb1_ref[...]
    o2_ref[...] = sc2 + b + sb2_ref[...]


def kernel(seq1, seq2, adj, msk, samp_bias1, samp_bias2,
           W_fc, gcn_bias, prelu_a, bilin_W, bilin_b):
    n = adj.shape[1]
    nh = W_fc.shape[1]
    nin = W_fc.shape[0]

    tb = 2000
    tm = 400
    td = 2000

    s1 = seq1[0]
    s2 = seq2[0]

    fts = pl.pallas_call(
        _fts_kernel,
        grid=(n // tb,),
        in_specs=[
            pl.BlockSpec((tb, nin), lambda i: (i, 0)),
            pl.BlockSpec((tb, nin), lambda i: (i, 0)),
            pl.BlockSpec((nin, nh), lambda i: (0, 0)),
        ],
        out_specs=pl.BlockSpec((tb, 2 * nh), lambda i: (i, 0)),
        out_shape=jax.ShapeDtypeStruct((n, 2 * nh), jnp.bfloat16),
    )(s1, s2, W_fc)

    bias2 = jnp.concatenate([gcn_bias, gcn_bias]).reshape(1, 2 * nh)
    a2 = prelu_a.reshape(1, 1)
    mskc = msk.reshape(n, 1)

    h, csum = pl.pallas_call(
        _gcn_kernel,
        grid=(n // tm,),
        in_specs=[
            pl.BlockSpec((1, tm, n), lambda i: (0, i, 0)),
            pl.BlockSpec((n, 2 * nh), lambda i: (0, 0)),
            pl.BlockSpec((tm, 1), lambda i: (i, 0)),
            pl.BlockSpec((1, 2 * nh), lambda i: (0, 0)),
            pl.BlockSpec((1, 1), lambda i: (0, 0)),
        ],
        out_specs=[
            pl.BlockSpec((tm, 2 * nh), lambda i: (i, 0)),
            pl.BlockSpec((1, nh), lambda i: (0, 0)),
        ],
        out_shape=[
            jax.ShapeDtypeStruct((n, 2 * nh), jnp.bfloat16),
            jax.ShapeDtypeStruct((1, nh), jnp.float32),
        ],
        compiler_params=pltpu.CompilerParams(
            vmem_limit_bytes=60 * 1024 * 1024,
        ),
    )(adj, fts, mskc, bias2, a2)

    wt = bilin_W.T
    b2 = bilin_b.reshape(1, 1)
    sb1 = samp_bias1.reshape(n, 1)
    sb2 = samp_bias2.reshape(n, 1)

    o1, o2 = pl.pallas_call(
        _disc_kernel,
        grid=(n // td,),
        in_specs=[
            pl.BlockSpec((td, 2 * nh), lambda i: (i, 0)),
            pl.BlockSpec((1, nh), lambda i: (0, 0)),
            pl.BlockSpec((1, n), lambda i: (0, 0)),
            pl.BlockSpec((nh, nh), lambda i: (0, 0)),
            pl.BlockSpec((1, 1), lambda i: (0, 0)),
            pl.BlockSpec((td, 1), lambda i: (i, 0)),
            pl.BlockSpec((td, 1), lambda i: (i, 0)),
        ],
        out_specs=[
            pl.BlockSpec((td, 1), lambda i: (i, 0)),
            pl.BlockSpec((td, 1), lambda i: (i, 0)),
        ],
        out_shape=[
            jax.ShapeDtypeStruct((n, 1), jnp.float32),
            jax.ShapeDtypeStruct((n, 1), jnp.float32),
        ],
    )(h, csum, msk, wt, b2, sb1, sb2)

    logits = jnp.concatenate([o1[:, 0], o2[:, 0]])[None, :]
    return logits


# speedup vs baseline: 1.5785x; 1.0274x over previous
import jax
import jax.numpy as jnp
from jax.experimental import pallas as pl
from jax.experimental.pallas import tpu as pltpu


def _fts_kernel(s1_ref, s2_ref, w_ref, out_ref):
    nh = w_ref.shape[1]
    w = w_ref[...].astype(jnp.bfloat16)
    f1 = jnp.dot(s1_ref[...].astype(jnp.bfloat16), w,
                 preferred_element_type=jnp.float32)
    f2 = jnp.dot(s2_ref[...].astype(jnp.bfloat16), w,
                 preferred_element_type=jnp.float32)
    out_ref[:, :nh] = f1.astype(jnp.bfloat16)
    out_ref[:, nh:] = f2.astype(jnp.bfloat16)


def _gcn_kernel(adja_ref, adjb_ref, fts_ref, mska_ref, mskb_ref,
                bias2_ref, a_ref, ha_ref, hb_ref, csum_ref):
    i = pl.program_id(0)
    nh = csum_ref.shape[1]
    a = a_ref[0, 0]
    fts = fts_ref[...]
    bias2 = bias2_ref[...]

    acca = jnp.dot(adja_ref[0].astype(jnp.bfloat16), fts,
                   preferred_element_type=jnp.float32)
    outa = acca + bias2
    ha = jnp.where(outa >= 0, outa, a * outa)
    ha_ref[...] = ha.astype(jnp.bfloat16)

    accb = jnp.dot(adjb_ref[0].astype(jnp.bfloat16), fts,
                   preferred_element_type=jnp.float32)
    outb = accb + bias2
    hb = jnp.where(outb >= 0, outb, a * outb)
    hb_ref[...] = hb.astype(jnp.bfloat16)

    @pl.when(i == 0)
    def _():
        csum_ref[...] = jnp.zeros_like(csum_ref)

    part = (jnp.sum(ha[:, :nh] * mska_ref[...], axis=0, keepdims=True)
            + jnp.sum(hb[:, :nh] * mskb_ref[...], axis=0, keepdims=True))
    csum_ref[...] += part


def _disc_kernel(ha_ref, hb_ref, csum_ref, msk_ref, wt_ref, b_ref,
                 sb1a_ref, sb1b_ref, sb2a_ref, sb2b_ref,
                 o1a_ref, o1b_ref, o2a_ref, o2b_ref):
    nh = csum_ref.shape[1]
    smsk = jnp.sum(msk_ref[...])
    c = jax.nn.sigmoid(csum_ref[...] / smsk)
    v = jnp.dot(c, wt_ref[...],
                preferred_element_type=jnp.float32)
    b = b_ref[0, 0]
    ha = ha_ref[...].astype(jnp.float32)
    hb = hb_ref[...].astype(jnp.float32)
    o1a_ref[...] = jnp.sum(ha[:, :nh] * v, axis=1, keepdims=True) + b + sb1a_ref[...]
    o2a_ref[...] = jnp.sum(ha[:, nh:] * v, axis=1, keepdims=True) + b + sb2a_ref[...]
    o1b_ref[...] = jnp.sum(hb[:, :nh] * v, axis=1, keepdims=True) + b + sb1b_ref[...]
    o2b_ref[...] = jnp.sum(hb[:, nh:] * v, axis=1, keepdims=True) + b + sb2b_ref[...]


def kernel(seq1, seq2, adj, msk, samp_bias1, samp_bias2,
           W_fc, gcn_bias, prelu_a, bilin_W, bilin_b):
    n = adj.shape[1]
    nh = W_fc.shape[1]
    nin = W_fc.shape[0]
    nhalf = n // 2

    tb = 2000
    tm = 200
    td = 1000

    s1 = seq1[0]
    s2 = seq2[0]

    fts = pl.pallas_call(
        _fts_kernel,
        grid=(n // tb,),
        in_specs=[
            pl.BlockSpec((tb, nin), lambda i: (i, 0)),
            pl.BlockSpec((tb, nin), lambda i: (i, 0)),
            pl.BlockSpec((nin, nh), lambda i: (0, 0)),
        ],
        out_specs=pl.BlockSpec((tb, 2 * nh), lambda i: (i, 0)),
        out_shape=jax.ShapeDtypeStruct((n, 2 * nh), jnp.bfloat16),
    )(s1, s2, W_fc)

    bias2 = jnp.concatenate([gcn_bias, gcn_bias]).reshape(1, 2 * nh)
    a2 = prelu_a.reshape(1, 1)
    mskc = msk.reshape(n, 1)
    nb = nhalf // tm

    ha, hb, csum = pl.pallas_call(
        _gcn_kernel,
        grid=(nb,),
        in_specs=[
            pl.BlockSpec((1, tm, n), lambda i: (0, i, 0)),
            pl.BlockSpec((1, tm, n), lambda i, _nb=nb: (0, i + _nb, 0)),
            pl.BlockSpec((n, 2 * nh), lambda i: (0, 0)),
            pl.BlockSpec((tm, 1), lambda i: (i, 0)),
            pl.BlockSpec((tm, 1), lambda i, _nb=nb: (i + _nb, 0)),
            pl.BlockSpec((1, 2 * nh), lambda i: (0, 0)),
            pl.BlockSpec((1, 1), lambda i: (0, 0)),
        ],
        out_specs=[
            pl.BlockSpec((tm, 2 * nh), lambda i: (i, 0)),
            pl.BlockSpec((tm, 2 * nh), lambda i: (i, 0)),
            pl.BlockSpec((1, nh), lambda i: (0, 0)),
        ],
        out_shape=[
            jax.ShapeDtypeStruct((nhalf, 2 * nh), jnp.bfloat16),
            jax.ShapeDtypeStruct((nhalf, 2 * nh), jnp.bfloat16),
            jax.ShapeDtypeStruct((1, nh), jnp.float32),
        ],
        compiler_params=pltpu.CompilerParams(
            vmem_limit_bytes=60 * 1024 * 1024,
        ),
    )(adj, adj, fts, mskc, mskc, bias2, a2)

    wt = bilin_W.T
    b2 = bilin_b.reshape(1, 1)
    sb1 = samp_bias1.reshape(n, 1)
    sb2 = samp_bias2.reshape(n, 1)
    ndb = nhalf // td

    col = lambda i: (i, 0)
    colb = lambda i, _nd=ndb: (i + _nd, 0)
    o1a, o1b, o2a, o2b = pl.pallas_call(
        _disc_kernel,
        grid=(ndb,),
        in_specs=[
            pl.BlockSpec((td, 2 * nh), col),
            pl.BlockSpec((td, 2 * nh), col),
            pl.BlockSpec((1, nh), lambda i: (0, 0)),
            pl.BlockSpec((1, n), lambda i: (0, 0)),
            pl.BlockSpec((nh, nh), lambda i: (0, 0)),
            pl.BlockSpec((1, 1), lambda i: (0, 0)),
            pl.BlockSpec((td, 1), col),
            pl.BlockSpec((td, 1), colb),
            pl.BlockSpec((td, 1), col),
            pl.BlockSpec((td, 1), colb),
        ],
        out_specs=[
            pl.BlockSpec((td, 1), col),
            pl.BlockSpec((td, 1), col),
            pl.BlockSpec((td, 1), col),
            pl.BlockSpec((td, 1), col),
        ],
        out_shape=[
            jax.ShapeDtypeStruct((nhalf, 1), jnp.float32),
            jax.ShapeDtypeStruct((nhalf, 1), jnp.float32),
            jax.ShapeDtypeStruct((nhalf, 1), jnp.float32),
            jax.ShapeDtypeStruct((nhalf, 1), jnp.float32),
        ],
    )(ha, hb, csum, msk, wt, b2, sb1, sb1, sb2, sb2)

    logits = jnp.concatenate(
        [o1a[:, 0], o1b[:, 0], o2a[:, 0], o2b[:, 0]])[None, :]
    return logits
